# Initial kernel scaffold; baseline (speedup 1.0000x reference)
import jax
import jax.numpy as jnp
from jax.experimental import pallas as pl


def kernel(nodes, edges, senders, receivers, globals_, n_node, n_edge, W_node, b_node, W_edge, b_edge, W_gn, b_gn, W_ge, b_ge, W_g, b_g, W_fg, b_fg):
    raise NotImplementedError("write your pallas kernel here")


# speedup vs baseline: 2.6703x; 2.6703x over previous
import functools

import jax
import jax.numpy as jnp
from jax import lax
from jax.experimental import pallas as pl
from jax.experimental.pallas import tpu as pltpu
from jax.experimental.pallas import tpu_sc as plsc

_N, _E, _D, _DE, _DG, _H, _GH = 10000, 320000, 128, 16, 16, 32, 16
_NC, _NS = 2, 16
_NW = _NC * _NS
_EW = _E // _NW
_CH = 80
_NCHUNK = _EW // _CH
_NPAD = 10240
_RPT = _NPAD // _NS
_EBLK = 4000

_PREC = lax.Precision.HIGHEST


def _leaky(x):
    return jnp.maximum(x, 0.01 * x)


def _node_proj_body(nodes_ref, wc_ref, ps_ref, pr_ref, nsum_ref):
    n = nodes_ref[...]
    p = jnp.dot(n, wc_ref[...], preferred_element_type=jnp.float32,
                precision=_PREC)
    ps_ref[...] = p[:, :2 * _H]
    pr_ref[...] = p[:, 2 * _H:]
    nsum_ref[...] = jnp.sum(n, axis=0, keepdims=True)


_node_proj = pl.pallas_call(
    _node_proj_body,
    out_shape=(jax.ShapeDtypeStruct((_N, 2 * _H), jnp.float32),
               jax.ShapeDtypeStruct((_N, 2 * _H), jnp.float32),
               jax.ShapeDtypeStruct((1, _D), jnp.float32)),
)


_EFOLD = 8
_ER = _E // _EFOLD
_ECOL = _EFOLD * _DE
_PCOL = _EFOLD * 2 * _H


def _edge_proj_body(g_ref, wgc_ref, bc_ref, e_ref, wblk_ref, pe_ref, esum_ref):
    i = pl.program_id(0)
    blk = e_ref[...]
    cvec = jnp.dot(g_ref[...], wgc_ref[...], preferred_element_type=jnp.float32,
                   precision=_PREC) + bc_ref[...]
    cfull = jnp.concatenate([cvec] * _EFOLD, axis=1)
    pe_ref[...] = jnp.dot(blk, wblk_ref[...], preferred_element_type=jnp.float32,
                          precision=_PREC) + cfull

    @pl.when(i == 0)
    def _init():
        esum_ref[...] = jnp.zeros_like(esum_ref)

    s128 = jnp.sum(blk, axis=0, keepdims=True)
    s = s128[:, :_DE]
    for j in range(1, _EFOLD):
        s = s + s128[:, j * _DE:(j + 1) * _DE]
    esum_ref[...] = esum_ref[...] + s


_edge_proj = pl.pallas_call(
    _edge_proj_body,
    grid=(_ER // _EBLK,),
    in_specs=[pl.BlockSpec((1, _DG), lambda i: (0, 0)),
              pl.BlockSpec((_DG, 2 * _H), lambda i: (0, 0)),
              pl.BlockSpec((1, 2 * _H), lambda i: (0, 0)),
              pl.BlockSpec((_EBLK, _ECOL), lambda i: (i, 0)),
              pl.BlockSpec((_ECOL, _PCOL), lambda i: (0, 0))],
    out_specs=(pl.BlockSpec((_EBLK, _PCOL), lambda i: (i, 0)),
               pl.BlockSpec((1, _DE), lambda i: (0, 0))),
    out_shape=(jax.ShapeDtypeStruct((_ER, _PCOL), jnp.float32),
               jax.ShapeDtypeStruct((1, _DE), jnp.float32)),
)


def _global_body(nsum_ref, esum_ref, g_ref, wgn_ref, bgn_ref, wge_ref,
                 bge_ref, wg_ref, bg_ref, wfg_ref, bfg_ref, out_ref):
    tn = _leaky(jnp.dot(nsum_ref[...], wgn_ref[...],
                        preferred_element_type=jnp.float32,
                        precision=_PREC) + bgn_ref[...])
    te = _leaky(jnp.dot(esum_ref[...], wge_ref[...],
                        preferred_element_type=jnp.float32,
                        precision=_PREC) + bge_ref[...])
    tg = _leaky(jnp.dot(g_ref[...], wg_ref[...],
                        preferred_element_type=jnp.float32,
                        precision=_PREC) + bg_ref[...])
    fa = jnp.concatenate([tg, tn, te], axis=1)
    out_ref[...] = _leaky(jnp.dot(fa, wfg_ref[...],
                                  preferred_element_type=jnp.float32,
                                  precision=_PREC) + bfg_ref[...])


_global_update = pl.pallas_call(
    _global_body,
    out_shape=jax.ShapeDtypeStruct((1, _GH), jnp.float32),
)


def _sc_body(ps_hbm, pr_hbm, pe_hbm, sidx_hbm, ridx_hbm, zeros_hbm,
             ne_hbm, part_hbm,
             sidx_v, ridx_v, ps_v, pr_v, pe_v, msg_v, eout_v,
             acc_sh, sem_ps, sem_pr, sem_pe):
    c = lax.axis_index("c")
    s = lax.axis_index("s")
    wid = c * _NS + s
    pltpu.sync_copy(zeros_hbm.at[pl.ds(s * _RPT, _RPT)],
                    acc_sh.at[pl.ds(s * _RPT, _RPT)])
    pltpu.sync_copy(sidx_hbm.at[wid], sidx_v)
    pltpu.sync_copy(ridx_hbm.at[wid], ridx_v)
    plsc.subcore_barrier()
    ebase = wid * _EW

    def chunk_body(j, carry):
        row0 = ebase + j * _CH
        cp_ps = pltpu.async_copy(ps_hbm.at[sidx_v.at[j]], ps_v, sem_ps)
        cp_pr = pltpu.async_copy(pr_hbm.at[ridx_v.at[j]], pr_v, sem_pr)
        cp_pe = pltpu.async_copy(pe_hbm.at[pl.ds(row0, _CH)], pe_v, sem_pe)
        cp_ps.wait()
        cp_pr.wait()
        cp_pe.wait()

        def edge_body(e, carry2):
            for g in range(4):
                sl = pl.ds(16 * g, 16)
                v = ps_v[e, sl] + pr_v[e, sl] + pe_v[e, sl]
                o = jnp.maximum(v, 0.01 * v)
                if g < 2:
                    msg_v[e, pl.ds(16 * g, 16)] = o
                else:
                    eout_v[e, pl.ds(16 * (g - 2), 16)] = o
            return carry2

        lax.fori_loop(0, _CH, edge_body, 0, unroll=2)
        pltpu.sync_copy(eout_v, ne_hbm.at[pl.ds(row0, _CH)])
        pltpu.sync_copy(msg_v, acc_sh.at[ridx_v.at[j]], add=True)
        return carry

    lax.fori_loop(0, _NCHUNK, chunk_body, 0)
    plsc.subcore_barrier()
    pltpu.sync_copy(acc_sh.at[pl.ds(s * _RPT, _RPT)],
                    part_hbm.at[c, pl.ds(s * _RPT, _RPT)])


_sc_edges = pl.kernel(
    _sc_body,
    out_type=(jax.ShapeDtypeStruct((_E, _H), jnp.float32),
              jax.ShapeDtypeStruct((_NC, _NPAD, _H), jnp.float32)),
    mesh=plsc.VectorSubcoreMesh(core_axis_name="c", subcore_axis_name="s"),
    compiler_params=pltpu.CompilerParams(use_tc_tiling_on_sc=False),
    scratch_types=[
        pltpu.VMEM((_NCHUNK, _CH), jnp.int32),
        pltpu.VMEM((_NCHUNK, _CH), jnp.int32),
        pltpu.VMEM((_CH, 4 * 16), jnp.float32),
        pltpu.VMEM((_CH, 4 * 16), jnp.float32),
        pltpu.VMEM((_CH, 4 * 16), jnp.float32),
        pltpu.VMEM((_CH, _H), jnp.float32),
        pltpu.VMEM((_CH, _H), jnp.float32),
        pltpu.VMEM_SHARED((_NPAD, _H), jnp.float32),
        pltpu.SemaphoreType.DMA,
        pltpu.SemaphoreType.DMA,
        pltpu.SemaphoreType.DMA,
    ],
)


def _combine_body(part_ref, out_ref):
    out_ref[...] = part_ref[0, :_N, :] + part_ref[1, :_N, :]


_combine = pl.pallas_call(
    _combine_body,
    out_shape=jax.ShapeDtypeStruct((_N, _H), jnp.float32),
)


def kernel(nodes, edges, senders, receivers, globals_, n_node, n_edge,
           W_node, b_node, W_edge, b_edge, W_gn, b_gn, W_ge, b_ge,
           W_g, b_g, W_fg, b_fg):
    senders = senders.astype(jnp.int32)
    receivers = receivers.astype(jnp.int32)
    wc = jnp.concatenate(
        [jnp.concatenate([W_node[:_D], W_edge[:_D]], axis=1),
         jnp.concatenate([W_node[_D:2 * _D], W_edge[_D:2 * _D]], axis=1)],
        axis=1)
    wec = jnp.concatenate([W_node[2 * _D:2 * _D + _DE],
                           W_edge[2 * _D:2 * _D + _DE]], axis=1)
    wgc = jnp.concatenate([W_node[2 * _D + _DE:],
                           W_edge[2 * _D + _DE:]], axis=1)
    bc = jnp.concatenate([b_node, b_edge]).reshape(1, 2 * _H)

    wblk = jnp.kron(jnp.eye(_EFOLD, dtype=jnp.float32), wec)
    ps, pr, nsum = _node_proj(nodes, wc)
    pe_r, esum = _edge_proj(globals_, wgc, bc, edges.reshape(_ER, _ECOL), wblk)
    pe = pe_r.reshape(_E, 2 * _H)
    new_global = _global_update(nsum, esum, globals_,
                                W_gn, b_gn.reshape(1, -1),
                                W_ge, b_ge.reshape(1, -1),
                                W_g, b_g.reshape(1, -1),
                                W_fg, b_fg.reshape(1, -1))

    sidx2 = senders.reshape(_NW, _NCHUNK, _CH)
    ridx2 = receivers.reshape(_NW, _NCHUNK, _CH)
    zeros = jnp.zeros((_NPAD, _H), jnp.float32)
    new_edges, part = _sc_edges(ps, pr, pe, sidx2, ridx2, zeros)
    new_nodes = _combine(part)
    return new_nodes, new_edges, new_global


# speedup vs baseline: 3.2612x; 1.2213x over previous
import functools

import jax
import jax.numpy as jnp
from jax import lax
from jax.experimental import pallas as pl
from jax.experimental.pallas import tpu as pltpu
from jax.experimental.pallas import tpu_sc as plsc

_N, _E, _D, _DE, _DG, _H, _GH = 10000, 320000, 128, 16, 16, 32, 16
_NC, _NS = 2, 16
_NW = _NC * _NS
_EW = _E // _NW
_CH = 80
_NCHUNK = _EW // _CH
_NPAD = 10240
_RPT = _NPAD // _NS
_EBLK = 4000

_PREC = lax.Precision.HIGHEST


def _leaky(x):
    return jnp.maximum(x, 0.01 * x)


def _node_proj_body(nodes_ref, wc_ref, ps_ref, pr_ref, nsum_ref):
    n = nodes_ref[...]
    p = jnp.dot(n, wc_ref[...], preferred_element_type=jnp.float32,
                precision=_PREC)
    ps_ref[...] = p[:, :2 * _H]
    pr_ref[...] = p[:, 2 * _H:]
    nsum_ref[...] = jnp.sum(n, axis=0, keepdims=True)


_node_proj = pl.pallas_call(
    _node_proj_body,
    out_shape=(jax.ShapeDtypeStruct((_N, 2 * _H), jnp.float32),
               jax.ShapeDtypeStruct((_N, 2 * _H), jnp.float32),
               jax.ShapeDtypeStruct((1, _D), jnp.float32)),
)


_EFOLD = 8
_ER = _E // _EFOLD
_ECOL = _EFOLD * _DE
_PCOL = _EFOLD * 2 * _H


def _edge_proj_body(g_ref, wgc_ref, bc_ref, e_ref, wblk_ref, pe_ref, esum_ref):
    i = pl.program_id(0)
    blk = e_ref[...]
    cvec = jnp.dot(g_ref[...], wgc_ref[...], preferred_element_type=jnp.float32,
                   precision=_PREC) + bc_ref[...]
    cfull = jnp.concatenate([cvec] * _EFOLD, axis=1)
    pe_ref[...] = jnp.dot(blk, wblk_ref[...], preferred_element_type=jnp.float32,
                          precision=_PREC) + cfull

    @pl.when(i == 0)
    def _init():
        esum_ref[...] = jnp.zeros_like(esum_ref)

    s128 = jnp.sum(blk, axis=0, keepdims=True)
    s = s128[:, :_DE]
    for j in range(1, _EFOLD):
        s = s + s128[:, j * _DE:(j + 1) * _DE]
    esum_ref[...] = esum_ref[...] + s


_edge_proj = pl.pallas_call(
    _edge_proj_body,
    grid=(_ER // _EBLK,),
    in_specs=[pl.BlockSpec((1, _DG), lambda i: (0, 0)),
              pl.BlockSpec((_DG, 2 * _H), lambda i: (0, 0)),
              pl.BlockSpec((1, 2 * _H), lambda i: (0, 0)),
              pl.BlockSpec((_EBLK, _ECOL), lambda i: (i, 0)),
              pl.BlockSpec((_ECOL, _PCOL), lambda i: (0, 0))],
    out_specs=(pl.BlockSpec((_EBLK, _PCOL), lambda i: (i, 0)),
               pl.BlockSpec((1, _DE), lambda i: (0, 0))),
    out_shape=(jax.ShapeDtypeStruct((_ER, _PCOL), jnp.float32),
               jax.ShapeDtypeStruct((1, _DE), jnp.float32)),
)


def _global_body(nsum_ref, esum_ref, g_ref, wgn_ref, bgn_ref, wge_ref,
                 bge_ref, wg_ref, bg_ref, wfg_ref, bfg_ref, out_ref):
    tn = _leaky(jnp.dot(nsum_ref[...], wgn_ref[...],
                        preferred_element_type=jnp.float32,
                        precision=_PREC) + bgn_ref[...])
    te = _leaky(jnp.dot(esum_ref[...], wge_ref[...],
                        preferred_element_type=jnp.float32,
                        precision=_PREC) + bge_ref[...])
    tg = _leaky(jnp.dot(g_ref[...], wg_ref[...],
                        preferred_element_type=jnp.float32,
                        precision=_PREC) + bg_ref[...])
    fa = jnp.concatenate([tg, tn, te], axis=1)
    out_ref[...] = _leaky(jnp.dot(fa, wfg_ref[...],
                                  preferred_element_type=jnp.float32,
                                  precision=_PREC) + bfg_ref[...])


_global_update = pl.pallas_call(
    _global_body,
    out_shape=jax.ShapeDtypeStruct((1, _GH), jnp.float32),
)


_PER = _CH // _EFOLD
_PEW = _EW // _EFOLD


def _sc_body(ps_hbm, pr_hbm, pe_hbm, sidx_hbm, ridx_hbm, zeros_hbm,
             ne_hbm, part_hbm,
             sidx_v, ridx_v,
             ps_a, pr_a, pe_a, msg_a, eout_a,
             ps_b, pr_b, pe_b, msg_b, eout_b,
             acc_sh,
             s1a, s2a, s3a, s4a, s5a,
             s1b, s2b, s3b, s4b, s5b):
    c_ax = lax.axis_index("c")
    s_ax = lax.axis_index("s")
    wid = c_ax * _NS + s_ax
    pltpu.sync_copy(zeros_hbm.at[pl.ds(s_ax * _RPT, _RPT)],
                    acc_sh.at[pl.ds(s_ax * _RPT, _RPT)])
    pltpu.sync_copy(sidx_hbm.at[wid], sidx_v)
    pltpu.sync_copy(ridx_hbm.at[wid], ridx_v)
    plsc.subcore_barrier()
    ebase = wid * _EW
    pebase = wid * _PEW

    bufs_a = (ps_a, pr_a, pe_a, msg_a, eout_a, s1a, s2a, s3a, s4a, s5a)
    bufs_b = (ps_b, pr_b, pe_b, msg_b, eout_b, s1b, s2b, s3b, s4b, s5b)

    def start_gathers(c, bufs):
        ps_t, pr_t, pe_t = bufs[0], bufs[1], bufs[2]
        s1, s2, s3 = bufs[5], bufs[6], bufs[7]
        pltpu.async_copy(ps_hbm.at[sidx_v.at[c]], ps_t, s1)
        pltpu.async_copy(pr_hbm.at[ridx_v.at[c]], pr_t, s2)
        pltpu.async_copy(pe_hbm.at[pl.ds(pebase + c * _PER, _PER)], pe_t, s3)

    def wait_gathers(c, bufs):
        ps_t, pr_t, pe_t = bufs[0], bufs[1], bufs[2]
        s1, s2, s3 = bufs[5], bufs[6], bufs[7]
        pltpu.make_async_copy(ps_hbm.at[sidx_v.at[c]], ps_t, s1).wait()
        pltpu.make_async_copy(pr_hbm.at[ridx_v.at[c]], pr_t, s2).wait()
        pltpu.make_async_copy(
            pe_hbm.at[pl.ds(pebase + c * _PER, _PER)], pe_t, s3).wait()

    def wait_stores(c, bufs):
        msg_t, eout_t = bufs[3], bufs[4]
        s4, s5 = bufs[8], bufs[9]
        pltpu.make_async_copy(
            eout_t, ne_hbm.at[pl.ds(ebase + c * _CH, _CH)], s4).wait()
        pltpu.make_async_copy(msg_t, acc_sh.at[ridx_v.at[c]], s5).wait()

    def compute(bufs):
        ps_t, pr_t, pe_t, msg_t, eout_t = bufs[:5]

        def row_body(er, carry):
            for sub in range(_EFOLD):
                e = er * _EFOLD + sub
                for g in range(4):
                    sl = pl.ds(16 * g, 16)
                    v = (ps_t[e, sl] + pr_t[e, sl]
                         + pe_t[er, pl.ds(sub * 64 + 16 * g, 16)])
                    o = jnp.maximum(v, 0.01 * v)
                    if g < 2:
                        msg_t[e, pl.ds(16 * g, 16)] = o
                    else:
                        eout_t[e, pl.ds(16 * (g - 2), 16)] = o
            return carry

        lax.fori_loop(0, _PER, row_body, 0)

    def chunk_step(c, bufs, first_pair):
        msg_t, eout_t = bufs[3], bufs[4]
        s4, s5 = bufs[8], bufs[9]
        wait_gathers(c, bufs)
        if first_pair is None:
            pl.when(c >= 2)(lambda: wait_stores(c - 2, bufs))
        elif not first_pair:
            wait_stores(c - 2, bufs)
        compute(bufs)
        pltpu.async_copy(eout_t, ne_hbm.at[pl.ds(ebase + c * _CH, _CH)], s4)
        pltpu.async_copy(msg_t, acc_sh.at[ridx_v.at[c]], s5, add=True)
        if isinstance(c, int):
            if c + 2 < _NCHUNK:
                start_gathers(c + 2, bufs)
        else:
            pl.when(c + 2 < _NCHUNK)(lambda: start_gathers(c + 2, bufs))

    start_gathers(0, bufs_a)
    start_gathers(1, bufs_b)

    def pair(j2, carry):
        c0 = j2 * 2
        chunk_step(c0, bufs_a, None)
        chunk_step(c0 + 1, bufs_b, None)
        return carry

    lax.fori_loop(0, (_NCHUNK - 1) // 2, pair, 0)
    chunk_step(_NCHUNK - 1, bufs_a, False)
    wait_stores(_NCHUNK - 2, bufs_b)
    wait_stores(_NCHUNK - 1, bufs_a)
    plsc.subcore_barrier()
    pltpu.sync_copy(acc_sh.at[pl.ds(s_ax * _RPT, _RPT)],
                    part_hbm.at[c_ax, pl.ds(s_ax * _RPT, _RPT)])


_sc_edges = pl.kernel(
    _sc_body,
    out_type=(jax.ShapeDtypeStruct((_E, _H), jnp.float32),
              jax.ShapeDtypeStruct((_NC, _NPAD, _H), jnp.float32)),
    mesh=plsc.VectorSubcoreMesh(core_axis_name="c", subcore_axis_name="s"),
    compiler_params=pltpu.CompilerParams(use_tc_tiling_on_sc=False),
    scratch_types=[
        pltpu.VMEM((_NCHUNK, _CH), jnp.int32),
        pltpu.VMEM((_NCHUNK, _CH), jnp.int32),
        pltpu.VMEM((_CH, 4 * 16), jnp.float32),
        pltpu.VMEM((_CH, 4 * 16), jnp.float32),
        pltpu.VMEM((_PER, _PCOL), jnp.float32),
        pltpu.VMEM((_CH, _H), jnp.float32),
        pltpu.VMEM((_CH, _H), jnp.float32),
        pltpu.VMEM((_CH, 4 * 16), jnp.float32),
        pltpu.VMEM((_CH, 4 * 16), jnp.float32),
        pltpu.VMEM((_PER, _PCOL), jnp.float32),
        pltpu.VMEM((_CH, _H), jnp.float32),
        pltpu.VMEM((_CH, _H), jnp.float32),
        pltpu.VMEM_SHARED((_NPAD, _H), jnp.float32),
        pltpu.SemaphoreType.DMA,
        pltpu.SemaphoreType.DMA,
        pltpu.SemaphoreType.DMA,
        pltpu.SemaphoreType.DMA,
        pltpu.SemaphoreType.DMA,
        pltpu.SemaphoreType.DMA,
        pltpu.SemaphoreType.DMA,
        pltpu.SemaphoreType.DMA,
        pltpu.SemaphoreType.DMA,
        pltpu.SemaphoreType.DMA,
    ],
)


def _combine_body(part_ref, out_ref):
    out_ref[...] = part_ref[0, :_N, :] + part_ref[1, :_N, :]


_combine = pl.pallas_call(
    _combine_body,
    out_shape=jax.ShapeDtypeStruct((_N, _H), jnp.float32),
)


def kernel(nodes, edges, senders, receivers, globals_, n_node, n_edge,
           W_node, b_node, W_edge, b_edge, W_gn, b_gn, W_ge, b_ge,
           W_g, b_g, W_fg, b_fg):
    senders = senders.astype(jnp.int32)
    receivers = receivers.astype(jnp.int32)
    wc = jnp.concatenate(
        [jnp.concatenate([W_node[:_D], W_edge[:_D]], axis=1),
         jnp.concatenate([W_node[_D:2 * _D], W_edge[_D:2 * _D]], axis=1)],
        axis=1)
    wec = jnp.concatenate([W_node[2 * _D:2 * _D + _DE],
                           W_edge[2 * _D:2 * _D + _DE]], axis=1)
    wgc = jnp.concatenate([W_node[2 * _D + _DE:],
                           W_edge[2 * _D + _DE:]], axis=1)
    bc = jnp.concatenate([b_node, b_edge]).reshape(1, 2 * _H)

    wblk = jnp.kron(jnp.eye(_EFOLD, dtype=jnp.float32), wec)
    ps, pr, nsum = _node_proj(nodes, wc)
    pe_r, esum = _edge_proj(globals_, wgc, bc, edges.reshape(_ER, _ECOL), wblk)
    new_global = _global_update(nsum, esum, globals_,
                                W_gn, b_gn.reshape(1, -1),
                                W_ge, b_ge.reshape(1, -1),
                                W_g, b_g.reshape(1, -1),
                                W_fg, b_fg.reshape(1, -1))

    sidx2 = senders.reshape(_NW, _NCHUNK, _CH)
    ridx2 = receivers.reshape(_NW, _NCHUNK, _CH)
    zeros = jnp.zeros((_NPAD, _H), jnp.float32)
    new_edges, part = _sc_edges(ps, pr, pe_r, sidx2, ridx2, zeros)
    new_nodes = _combine(part)
    return new_nodes, new_edges, new_global


# speedup vs baseline: 3.2617x; 1.0001x over previous
import functools

import jax
import jax.numpy as jnp
from jax import lax
from jax.experimental import pallas as pl
from jax.experimental.pallas import tpu as pltpu
from jax.experimental.pallas import tpu_sc as plsc

_N, _E, _D, _DE, _DG, _H, _GH = 10000, 320000, 128, 16, 16, 32, 16
_NC, _NS = 2, 16
_NW = _NC * _NS
_EW = _E // _NW
_CH = 80
_NCHUNK = _EW // _CH
_NPAD = 10240
_RPT = _NPAD // _NS
_EBLK = 4000

_PREC = lax.Precision.HIGHEST


def _leaky(x):
    return jnp.maximum(x, 0.01 * x)


def _node_proj_body(nodes_ref, wc_ref, ps_ref, pr_ref, nsum_ref):
    n = nodes_ref[...]
    p = jnp.dot(n, wc_ref[...], preferred_element_type=jnp.float32,
                precision=_PREC)
    ps_ref[...] = p[:, :2 * _H]
    pr_ref[...] = p[:, 2 * _H:]
    nsum_ref[...] = jnp.sum(n, axis=0, keepdims=True)


_node_proj = pl.pallas_call(
    _node_proj_body,
    out_shape=(jax.ShapeDtypeStruct((_N, 2 * _H), jnp.float32),
               jax.ShapeDtypeStruct((_N, 2 * _H), jnp.float32),
               jax.ShapeDtypeStruct((1, _D), jnp.float32)),
)


_EFOLD = 8
_ER = _E // _EFOLD
_ECOL = _EFOLD * _DE
_PCOL = _EFOLD * 2 * _H


def _edge_proj_body(g_ref, wgc_ref, bc_ref, e_ref, wblk_ref, pe_ref, esum_ref):
    i = pl.program_id(0)
    blk = e_ref[...]
    cvec = jnp.dot(g_ref[...], wgc_ref[...], preferred_element_type=jnp.float32,
                   precision=_PREC) + bc_ref[...]
    cfull = jnp.concatenate([cvec] * _EFOLD, axis=1)
    pe_ref[...] = jnp.dot(blk, wblk_ref[...], preferred_element_type=jnp.float32,
                          precision=_PREC) + cfull

    @pl.when(i == 0)
    def _init():
        esum_ref[...] = jnp.zeros_like(esum_ref)

    s128 = jnp.sum(blk, axis=0, keepdims=True)
    s = s128[:, :_DE]
    for j in range(1, _EFOLD):
        s = s + s128[:, j * _DE:(j + 1) * _DE]
    esum_ref[...] = esum_ref[...] + s


_edge_proj = pl.pallas_call(
    _edge_proj_body,
    grid=(_ER // _EBLK,),
    compiler_params=pltpu.CompilerParams(
        allow_input_fusion=[False, False, False, True, False]),
    in_specs=[pl.BlockSpec((1, _DG), lambda i: (0, 0)),
              pl.BlockSpec((_DG, 2 * _H), lambda i: (0, 0)),
              pl.BlockSpec((1, 2 * _H), lambda i: (0, 0)),
              pl.BlockSpec((_EBLK, _ECOL), lambda i: (i, 0)),
              pl.BlockSpec((_ECOL, _PCOL), lambda i: (0, 0))],
    out_specs=(pl.BlockSpec((_EBLK, _PCOL), lambda i: (i, 0)),
               pl.BlockSpec((1, _DE), lambda i: (0, 0))),
    out_shape=(jax.ShapeDtypeStruct((_ER, _PCOL), jnp.float32),
               jax.ShapeDtypeStruct((1, _DE), jnp.float32)),
)


def _global_body(nsum_ref, esum_ref, g_ref, wgn_ref, bgn_ref, wge_ref,
                 bge_ref, wg_ref, bg_ref, wfg_ref, bfg_ref, out_ref):
    tn = _leaky(jnp.dot(nsum_ref[...], wgn_ref[...],
                        preferred_element_type=jnp.float32,
                        precision=_PREC) + bgn_ref[...])
    te = _leaky(jnp.dot(esum_ref[...], wge_ref[...],
                        preferred_element_type=jnp.float32,
                        precision=_PREC) + bge_ref[...])
    tg = _leaky(jnp.dot(g_ref[...], wg_ref[...],
                        preferred_element_type=jnp.float32,
                        precision=_PREC) + bg_ref[...])
    fa = jnp.concatenate([tg, tn, te], axis=1)
    out_ref[...] = _leaky(jnp.dot(fa, wfg_ref[...],
                                  preferred_element_type=jnp.float32,
                                  precision=_PREC) + bfg_ref[...])


_global_update = pl.pallas_call(
    _global_body,
    out_shape=jax.ShapeDtypeStruct((1, _GH), jnp.float32),
)


_PER = _CH // _EFOLD
_PEW = _EW // _EFOLD


def _sc_body(ps_hbm, pr_hbm, pe_hbm, sidx_hbm, ridx_hbm, zeros_hbm,
             ne_hbm, part_hbm,
             sidx_v, ridx_v,
             ps_a, pr_a, pe_a, msg_a, eout_a,
             ps_b, pr_b, pe_b, msg_b, eout_b,
             acc_sh,
             s1a, s2a, s3a, s4a, s5a,
             s1b, s2b, s3b, s4b, s5b):
    c_ax = lax.axis_index("c")
    s_ax = lax.axis_index("s")
    wid = c_ax * _NS + s_ax
    pltpu.sync_copy(zeros_hbm.at[pl.ds(s_ax * _RPT, _RPT)],
                    acc_sh.at[pl.ds(s_ax * _RPT, _RPT)])
    pltpu.sync_copy(sidx_hbm.at[pl.ds(wid * _EW, _EW)], sidx_v)
    pltpu.sync_copy(ridx_hbm.at[wid], ridx_v)
    plsc.subcore_barrier()
    ebase = wid * _EW
    pebase = wid * _PEW

    bufs_a = (ps_a, pr_a, pe_a, msg_a, eout_a, s1a, s2a, s3a, s4a, s5a)
    bufs_b = (ps_b, pr_b, pe_b, msg_b, eout_b, s1b, s2b, s3b, s4b, s5b)

    def start_gathers(c, bufs):
        ps_t, pr_t, pe_t = bufs[0], bufs[1], bufs[2]
        s1, s2, s3 = bufs[5], bufs[6], bufs[7]
        pltpu.async_copy(ps_hbm.at[sidx_v.at[pl.ds(c * _CH, _CH)]], ps_t, s1)
        pltpu.async_copy(pr_hbm.at[ridx_v.at[c]], pr_t, s2)
        pltpu.async_copy(pe_hbm.at[pl.ds(pebase + c * _PER, _PER)], pe_t, s3)

    def wait_gathers(c, bufs):
        ps_t, pr_t, pe_t = bufs[0], bufs[1], bufs[2]
        s1, s2, s3 = bufs[5], bufs[6], bufs[7]
        pltpu.make_async_copy(ps_hbm.at[sidx_v.at[pl.ds(c * _CH, _CH)]], ps_t, s1).wait()
        pltpu.make_async_copy(pr_hbm.at[ridx_v.at[c]], pr_t, s2).wait()
        pltpu.make_async_copy(
            pe_hbm.at[pl.ds(pebase + c * _PER, _PER)], pe_t, s3).wait()

    def wait_stores(c, bufs):
        msg_t, eout_t = bufs[3], bufs[4]
        s4, s5 = bufs[8], bufs[9]
        pltpu.make_async_copy(
            eout_t, ne_hbm.at[pl.ds(ebase + c * _CH, _CH)], s4).wait()
        pltpu.make_async_copy(msg_t, acc_sh.at[ridx_v.at[c]], s5).wait()

    def compute(bufs):
        ps_t, pr_t, pe_t, msg_t, eout_t = bufs[:5]

        def row_body(er, carry):
            for sub in range(_EFOLD):
                e = er * _EFOLD + sub
                for g in range(4):
                    sl = pl.ds(16 * g, 16)
                    v = (ps_t[e, sl] + pr_t[e, sl]
                         + pe_t[er, pl.ds(sub * 64 + 16 * g, 16)])
                    o = jnp.maximum(v, 0.01 * v)
                    if g < 2:
                        msg_t[e, pl.ds(16 * g, 16)] = o
                    else:
                        eout_t[e, pl.ds(16 * (g - 2), 16)] = o
            return carry

        lax.fori_loop(0, _PER, row_body, 0)

    def chunk_step(c, bufs, first_pair):
        msg_t, eout_t = bufs[3], bufs[4]
        s4, s5 = bufs[8], bufs[9]
        wait_gathers(c, bufs)
        if first_pair is None:
            pl.when(c >= 2)(lambda: wait_stores(c - 2, bufs))
        elif not first_pair:
            wait_stores(c - 2, bufs)
        compute(bufs)
        pltpu.async_copy(eout_t, ne_hbm.at[pl.ds(ebase + c * _CH, _CH)], s4)
        pltpu.async_copy(msg_t, acc_sh.at[ridx_v.at[c]], s5, add=True)
        if isinstance(c, int):
            if c + 2 < _NCHUNK:
                start_gathers(c + 2, bufs)
        else:
            pl.when(c + 2 < _NCHUNK)(lambda: start_gathers(c + 2, bufs))

    start_gathers(0, bufs_a)
    start_gathers(1, bufs_b)

    def pair(j2, carry):
        c0 = j2 * 2
        chunk_step(c0, bufs_a, None)
        chunk_step(c0 + 1, bufs_b, None)
        return carry

    lax.fori_loop(0, (_NCHUNK - 1) // 2, pair, 0)
    chunk_step(_NCHUNK - 1, bufs_a, False)
    wait_stores(_NCHUNK - 2, bufs_b)
    wait_stores(_NCHUNK - 1, bufs_a)
    plsc.subcore_barrier()
    pltpu.sync_copy(acc_sh.at[pl.ds(s_ax * _RPT, _RPT)],
                    part_hbm.at[c_ax, pl.ds(s_ax * _RPT, _RPT)])


_sc_edges = pl.kernel(
    _sc_body,
    out_type=(jax.ShapeDtypeStruct((_E, _H), jnp.float32),
              jax.ShapeDtypeStruct((_NC, _NPAD, _H), jnp.float32)),
    mesh=plsc.VectorSubcoreMesh(core_axis_name="c", subcore_axis_name="s"),
    compiler_params=pltpu.CompilerParams(use_tc_tiling_on_sc=False),
    scratch_types=[
        pltpu.VMEM((_EW,), jnp.int32),
        pltpu.VMEM((_NCHUNK, _CH), jnp.int32),
        pltpu.VMEM((_CH, 4 * 16), jnp.float32),
        pltpu.VMEM((_CH, 4 * 16), jnp.float32),
        pltpu.VMEM((_PER, _PCOL), jnp.float32),
        pltpu.VMEM((_CH, _H), jnp.float32),
        pltpu.VMEM((_CH, _H), jnp.float32),
        pltpu.VMEM((_CH, 4 * 16), jnp.float32),
        pltpu.VMEM((_CH, 4 * 16), jnp.float32),
        pltpu.VMEM((_PER, _PCOL), jnp.float32),
        pltpu.VMEM((_CH, _H), jnp.float32),
        pltpu.VMEM((_CH, _H), jnp.float32),
        pltpu.VMEM_SHARED((_NPAD, _H), jnp.float32),
        pltpu.SemaphoreType.DMA,
        pltpu.SemaphoreType.DMA,
        pltpu.SemaphoreType.DMA,
        pltpu.SemaphoreType.DMA,
        pltpu.SemaphoreType.DMA,
        pltpu.SemaphoreType.DMA,
        pltpu.SemaphoreType.DMA,
        pltpu.SemaphoreType.DMA,
        pltpu.SemaphoreType.DMA,
        pltpu.SemaphoreType.DMA,
    ],
)


def _combine_body(part_ref, out_ref):
    out_ref[...] = part_ref[0, :_N, :] + part_ref[1, :_N, :]


_combine = pl.pallas_call(
    _combine_body,
    out_shape=jax.ShapeDtypeStruct((_N, _H), jnp.float32),
)


def kernel(nodes, edges, senders, receivers, globals_, n_node, n_edge,
           W_node, b_node, W_edge, b_edge, W_gn, b_gn, W_ge, b_ge,
           W_g, b_g, W_fg, b_fg):
    senders = senders.astype(jnp.int32)
    receivers = receivers.astype(jnp.int32)
    wc = jnp.concatenate(
        [jnp.concatenate([W_node[:_D], W_edge[:_D]], axis=1),
         jnp.concatenate([W_node[_D:2 * _D], W_edge[_D:2 * _D]], axis=1)],
        axis=1)
    wec = jnp.concatenate([W_node[2 * _D:2 * _D + _DE],
                           W_edge[2 * _D:2 * _D + _DE]], axis=1)
    wgc = jnp.concatenate([W_node[2 * _D + _DE:],
                           W_edge[2 * _D + _DE:]], axis=1)
    bc = jnp.concatenate([b_node, b_edge]).reshape(1, 2 * _H)

    wblk = jnp.kron(jnp.eye(_EFOLD, dtype=jnp.float32), wec)
    ps, pr, nsum = _node_proj(nodes, wc)
    pe_r, esum = _edge_proj(globals_, wgc, bc, edges.reshape(_ER, _ECOL), wblk)
    new_global = _global_update(nsum, esum, globals_,
                                W_gn, b_gn.reshape(1, -1),
                                W_ge, b_ge.reshape(1, -1),
                                W_g, b_g.reshape(1, -1),
                                W_fg, b_fg.reshape(1, -1))

    ridx2 = receivers.reshape(_NW, _NCHUNK, _CH)
    zeros = jnp.zeros((_NPAD, _H), jnp.float32)
    new_edges, part = _sc_edges(ps, pr, pe_r, senders, ridx2, zeros)
    new_nodes = _combine(part)
    return new_nodes, new_edges, new_global
